# baseline (speedup 1.0000x reference)
import functools

import jax
import jax.numpy as jnp
from jax import lax
from jax.experimental import pallas as pl
from jax.experimental.pallas import tpu as pltpu
from jax.experimental.pallas import tpu_sc as plsc

B = 4096
L = 200
N = B * L
CHAR_DIM = 64
SEG_DIM = 32
BICHAR_DIM = 64
OUT_DIM = CHAR_DIM + SEG_DIM + BICHAR_DIM
WIDE = 128
LANES = 16

NUM_CORES = 2
NUM_SUBCORES = 16
NW = NUM_CORES * NUM_SUBCORES
TOK_PER_W = N // NW

CHUNK = 80
N_CHUNKS = TOK_PER_W // CHUNK


def _mix_embed_sc(flat_c, flat_b, flat_s, char_W, bichar_W, seg_W):
    mesh = plsc.VectorSubcoreMesh(core_axis_name="c", subcore_axis_name="s")

    @functools.partial(
        pl.kernel,
        mesh=mesh,
        out_type=jax.ShapeDtypeStruct((B, L, OUT_DIM), jnp.float32),
        scratch_types=[
            pltpu.VMEM((2, CHUNK), jnp.int32),
            pltpu.VMEM((2, CHUNK), jnp.int32),
            pltpu.VMEM((2, CHUNK), jnp.int32),
            pltpu.VMEM((2, CHUNK, WIDE), jnp.float32),
            pltpu.VMEM((2, CHUNK, WIDE), jnp.float32),
            pltpu.VMEM((2, CHUNK, WIDE), jnp.float32),
            pltpu.VMEM((2, CHUNK, OUT_DIM), jnp.float32),
            pltpu.SemaphoreType.DMA((2,)),
            pltpu.SemaphoreType.DMA((2,)),
            pltpu.SemaphoreType.DMA((2,)),
        ],
    )
    def kern(chars_hbm, bichars_hbm, segs_hbm, charw_hbm, bicharw_hbm,
             segw_hbm, out3_hbm, cidx_v, sidx_v, bidx_v, crow_v, srow_v,
             brow_v, row_v, sem_i, sem_g, sem_o):
        out_hbm = out3_hbm.reshape(N, OUT_DIM)
        wid = lax.axis_index("s") * NUM_CORES + lax.axis_index("c")
        wbase = wid * TOK_PER_W

        def issue_idx(chunk, bank):
            base = wbase + chunk * CHUNK
            pltpu.async_copy(chars_hbm.at[pl.ds(base, CHUNK)],
                             cidx_v.at[bank], sem_i.at[bank])
            pltpu.async_copy(segs_hbm.at[pl.ds(base, CHUNK)],
                             sidx_v.at[bank], sem_i.at[bank])
            pltpu.async_copy(bichars_hbm.at[pl.ds(base, CHUNK)],
                             bidx_v.at[bank], sem_i.at[bank])

        def wait_idx(bank):
            for _ in range(3):
                pltpu.make_async_copy(
                    chars_hbm.at[pl.ds(0, CHUNK)], cidx_v.at[bank],
                    sem_i.at[bank]).wait()

        def issue_gathers(bank):
            pltpu.async_copy(charw_hbm.at[cidx_v.at[bank]], crow_v.at[bank],
                             sem_g.at[bank])
            pltpu.async_copy(segw_hbm.at[sidx_v.at[bank]], srow_v.at[bank],
                             sem_g.at[bank])
            pltpu.async_copy(bicharw_hbm.at[bidx_v.at[bank]],
                             brow_v.at[bank], sem_g.at[bank])

        def wait_gathers(bank):
            pltpu.make_async_copy(
                charw_hbm.at[pl.ds(0, CHUNK)], crow_v.at[bank],
                sem_g.at[bank]).wait()
            pltpu.make_async_copy(
                segw_hbm.at[pl.ds(0, CHUNK)], srow_v.at[bank],
                sem_g.at[bank]).wait()
            pltpu.make_async_copy(
                bicharw_hbm.at[pl.ds(0, CHUNK)], brow_v.at[bank],
                sem_g.at[bank]).wait()

        def assemble(bank):
            def per_token(i, carry2):
                for k in range(CHAR_DIM // LANES):
                    row_v[bank, i, pl.ds(k * LANES, LANES)] = (
                        crow_v[bank, i, pl.ds(k * LANES, LANES)])
                for k in range(SEG_DIM // LANES):
                    row_v[bank, i, pl.ds(CHAR_DIM + k * LANES, LANES)] = (
                        srow_v[bank, i, pl.ds(k * LANES, LANES)])
                for k in range(BICHAR_DIM // LANES):
                    row_v[bank, i,
                          pl.ds(CHAR_DIM + SEG_DIM + k * LANES,
                                LANES)] = brow_v[bank, i,
                                                 pl.ds(k * LANES, LANES)]
                return carry2

            lax.fori_loop(0, CHUNK, per_token, 0)

        def issue_out(chunk, bank):
            base = wbase + chunk * CHUNK
            pltpu.async_copy(
                row_v.at[bank], out_hbm.at[pl.ds(base, CHUNK)],
                sem_o.at[bank])

        def wait_out(bank):
            pltpu.make_async_copy(
                out_hbm.at[pl.ds(0, CHUNK)], row_v.at[bank],
                sem_o.at[bank]).wait()

        issue_idx(0, 0)
        issue_idx(1, 1)
        wait_idx(0)
        issue_gathers(0)

        def steady(g, carry):
            q = lax.rem(g, 2)
            qn = 1 - q
            wait_gathers(q)
            wait_idx(qn)
            issue_gathers(qn)

            @pl.when(g >= 2)
            def _():
                wait_out(q)

            assemble(q)
            issue_idx(g + 2, q)
            issue_out(g, q)
            return carry

        lax.fori_loop(0, N_CHUNKS - 2, steady, 0)

        for g in (N_CHUNKS - 2, N_CHUNKS - 1):
            q = g % 2
            wait_gathers(q)
            if g == N_CHUNKS - 2:
                wait_idx(1 - q)
                issue_gathers(1 - q)
            wait_out(q)
            assemble(q)
            issue_out(g, q)
        wait_out(N_CHUNKS % 2)
        wait_out(1 - (N_CHUNKS % 2))

    return kern(flat_c, flat_b, flat_s, char_W, bichar_W, seg_W)


@jax.jit
def kernel(pad_chars, pad_bichars, pad_segs, char_W, bichar_W, seg_W):
    flat_c = pad_chars.reshape(-1).astype(jnp.int32)
    flat_b = pad_bichars.reshape(-1).astype(jnp.int32)
    flat_s = pad_segs.reshape(-1).astype(jnp.int32)
    char_W2 = jnp.tile(char_W, (1, 2))
    bichar_W2 = jnp.tile(bichar_W, (1, 2))
    seg_W4 = jnp.tile(seg_W, (1, 4))
    return _mix_embed_sc(flat_c, flat_b, flat_s, char_W2, bichar_W2, seg_W4)


# speedup vs baseline: 1.0864x; 1.0864x over previous
import functools

import jax
import jax.numpy as jnp
from jax import lax
from jax.experimental import pallas as pl
from jax.experimental.pallas import tpu as pltpu
from jax.experimental.pallas import tpu_sc as plsc

B = 4096
L = 200
N = B * L
CHAR_DIM = 64
SEG_DIM = 32
BICHAR_DIM = 64
OUT_DIM = CHAR_DIM + SEG_DIM + BICHAR_DIM
WIDE = 128
LANES = 16

NUM_CORES = 2
NUM_SUBCORES = 16
NW = NUM_CORES * NUM_SUBCORES
TOK_PER_W = N // NW

CHUNK = 80
N_CHUNKS = TOK_PER_W // CHUNK


def _mix_embed_sc(flat_c, flat_b, flat_s, char_W, bichar_W, seg_W):
    mesh = plsc.VectorSubcoreMesh(core_axis_name="c", subcore_axis_name="s")

    @functools.partial(
        pl.kernel,
        mesh=mesh,
        out_type=jax.ShapeDtypeStruct((N, OUT_DIM), jnp.float32),
        scratch_types=[
            pltpu.VMEM((2, CHUNK), jnp.int32),
            pltpu.VMEM((2, CHUNK), jnp.int32),
            pltpu.VMEM((2, CHUNK), jnp.int32),
            pltpu.VMEM((2, CHUNK, WIDE), jnp.float32),
            pltpu.VMEM((2, CHUNK, WIDE), jnp.float32),
            pltpu.VMEM((2, CHUNK, WIDE), jnp.float32),
            pltpu.VMEM((2, CHUNK, OUT_DIM), jnp.float32),
            pltpu.SemaphoreType.DMA((2,)),
            pltpu.SemaphoreType.DMA((2,)),
            pltpu.SemaphoreType.DMA((2,)),
        ],
    )
    def kern(chars_hbm, bichars_hbm, segs_hbm, charw_hbm, bicharw_hbm,
             segw_hbm, out_hbm, cidx_v, sidx_v, bidx_v, crow_v, srow_v,
             brow_v, row_v, sem_i, sem_g, sem_o):
        wid = lax.axis_index("s") * NUM_CORES + lax.axis_index("c")
        wbase = wid * TOK_PER_W

        def issue_idx(chunk, bank):
            base = wbase + chunk * CHUNK
            pltpu.async_copy(chars_hbm.at[pl.ds(base, CHUNK)],
                             cidx_v.at[bank], sem_i.at[bank])
            pltpu.async_copy(segs_hbm.at[pl.ds(base, CHUNK)],
                             sidx_v.at[bank], sem_i.at[bank])
            pltpu.async_copy(bichars_hbm.at[pl.ds(base, CHUNK)],
                             bidx_v.at[bank], sem_i.at[bank])

        def wait_idx(bank):
            for _ in range(3):
                pltpu.make_async_copy(
                    chars_hbm.at[pl.ds(0, CHUNK)], cidx_v.at[bank],
                    sem_i.at[bank]).wait()

        def issue_gathers(bank):
            pltpu.async_copy(charw_hbm.at[cidx_v.at[bank]], crow_v.at[bank],
                             sem_g.at[bank])
            pltpu.async_copy(segw_hbm.at[sidx_v.at[bank]], srow_v.at[bank],
                             sem_g.at[bank])
            pltpu.async_copy(bicharw_hbm.at[bidx_v.at[bank]],
                             brow_v.at[bank], sem_g.at[bank])

        def wait_gathers(bank):
            pltpu.make_async_copy(
                charw_hbm.at[pl.ds(0, CHUNK)], crow_v.at[bank],
                sem_g.at[bank]).wait()
            pltpu.make_async_copy(
                segw_hbm.at[pl.ds(0, CHUNK)], srow_v.at[bank],
                sem_g.at[bank]).wait()
            pltpu.make_async_copy(
                bicharw_hbm.at[pl.ds(0, CHUNK)], brow_v.at[bank],
                sem_g.at[bank]).wait()

        def assemble(bank):
            def per_token(i, carry2):
                for k in range(CHAR_DIM // LANES):
                    row_v[bank, i, pl.ds(k * LANES, LANES)] = (
                        crow_v[bank, i, pl.ds(k * LANES, LANES)])
                for k in range(SEG_DIM // LANES):
                    row_v[bank, i, pl.ds(CHAR_DIM + k * LANES, LANES)] = (
                        srow_v[bank, i, pl.ds(k * LANES, LANES)])
                for k in range(BICHAR_DIM // LANES):
                    row_v[bank, i,
                          pl.ds(CHAR_DIM + SEG_DIM + k * LANES,
                                LANES)] = brow_v[bank, i,
                                                 pl.ds(k * LANES, LANES)]
                return carry2

            lax.fori_loop(0, CHUNK, per_token, 0)

        def issue_out(chunk, bank):
            base = wbase + chunk * CHUNK
            pltpu.async_copy(
                row_v.at[bank], out_hbm.at[pl.ds(base, CHUNK)],
                sem_o.at[bank])

        def wait_out(bank):
            pltpu.make_async_copy(
                out_hbm.at[pl.ds(0, CHUNK)], row_v.at[bank],
                sem_o.at[bank]).wait()

        issue_idx(0, 0)
        issue_idx(1, 1)
        wait_idx(0)
        issue_gathers(0)

        def steady(g, carry):
            q = lax.rem(g, 2)
            qn = 1 - q
            wait_gathers(q)
            wait_idx(qn)
            issue_gathers(qn)

            @pl.when(g >= 2)
            def _():
                wait_out(q)

            assemble(q)
            issue_idx(g + 2, q)
            issue_out(g, q)
            return carry

        lax.fori_loop(0, N_CHUNKS - 2, steady, 0)

        for g in (N_CHUNKS - 2, N_CHUNKS - 1):
            q = g % 2
            wait_gathers(q)
            if g == N_CHUNKS - 2:
                wait_idx(1 - q)
                issue_gathers(1 - q)
            wait_out(q)
            assemble(q)
            issue_out(g, q)
        wait_out(N_CHUNKS % 2)
        wait_out(1 - (N_CHUNKS % 2))

    return kern(flat_c, flat_b, flat_s, char_W, bichar_W, seg_W)


@jax.jit
def kernel(pad_chars, pad_bichars, pad_segs, char_W, bichar_W, seg_W):
    flat_c = pad_chars.reshape(-1).astype(jnp.int32)
    flat_b = pad_bichars.reshape(-1).astype(jnp.int32)
    flat_s = pad_segs.reshape(-1).astype(jnp.int32)
    char_W2 = jnp.concatenate([char_W, char_W], axis=1)
    bichar_W2 = jnp.concatenate([bichar_W, bichar_W], axis=1)
    seg_W4 = jnp.concatenate([seg_W, seg_W, seg_W, seg_W], axis=1)
    out = _mix_embed_sc(flat_c, flat_b, flat_s, char_W2, bichar_W2, seg_W4)
    return out.reshape(B, L, OUT_DIM)
